# Initial kernel scaffold; baseline (speedup 1.0000x reference)
import jax
import jax.numpy as jnp
from jax.experimental import pallas as pl


def kernel(x, edge_index, edge_attr, batch, atom_emb, bond_emb, conv_eps, conv_W1, conv_b1, mlp_bn_g, mlp_bn_b, conv_W2, conv_b2, bn_g, bn_b, vn_emb0, vn_W1, vn_b1, vn_bn1_g, vn_bn1_b, vn_W2, vn_b2, vn_bn2_g, vn_bn2_b, pred_W, pred_b):
    raise NotImplementedError("write your pallas kernel here")


# speedup vs baseline: 1.7751x; 1.7751x over previous
import functools

import jax
import jax.numpy as jnp
from jax import lax
from jax.experimental import pallas as pl
from jax.experimental.pallas import tpu as pltpu
from jax.experimental.pallas import tpu_sc as plsc

N = 10000
E = 160000
G = 64
D = 300
H = 600
L = 5
T = 128
AV = 100
BV = 10

DH = 160
HP = 640
NB = 10
RB = N // NB
ATR = 1024
NC = 1000

CH = 80
NSUB = 16
NH = N // 2
TRIPS = 68
NCHP = NSUB * TRIPS
CAP = NCHP * CH
EPAD = 2 * CAP
STR = 320
NPH = NSUB * STR
GARB = NPH - 8
ZR = 160

_f32 = jnp.float32


def _dot(a, b):
    return jnp.dot(a, b, preferred_element_type=_f32)


def _sc_edge_body(hi_hbm, ctab_hbm, src_hbm, dst_hbm, cidx_hbm, out_hbm,
                  srcv, dstv, cidxv, rows, crows, zbuf, aggr_sh,
                  sem1, sem2):
    c = lax.axis_index("c")
    s = lax.axis_index("s")

    def zfill(i, carry):
        for k in range(DH // 16):
            zbuf[i, pl.ds(k * 16, 16)] = jnp.zeros((16,), _f32)
        return carry
    lax.fori_loop(0, ZR, zfill, 0)

    coff_hi = c * N
    coff_ct = c * NC

    for p in range(2):
        pltpu.sync_copy(zbuf, aggr_sh.at[pl.ds(s * STR, ZR)])
        pltpu.sync_copy(zbuf, aggr_sh.at[pl.ds(s * STR + ZR, STR - ZR)])
        plsc.subcore_barrier()

        pbase = p * NH

        def chunk(i, carry):
            base = (p * NCHP + s + i * NSUB) * CH
            pltpu.sync_copy(src_hbm.at[pl.ds(base, CH)], srcv)
            pltpu.sync_copy(cidx_hbm.at[pl.ds(base, CH)], cidxv)
            pltpu.sync_copy(dst_hbm.at[pl.ds(base, CH)], dstv)
            for k in range(CH // 16):
                sl = pl.ds(k * 16, 16)
                srcv[sl] = srcv[sl] + coff_hi
                cidxv[sl] = cidxv[sl] + coff_ct
                d = dstv[sl] - pbase
                ok = (d >= 0) & (d < NH)
                dstv[sl] = jnp.where(ok, d, GARB)
            cp1 = pltpu.async_copy(hi_hbm.at[srcv], rows, sem1)
            cp2 = pltpu.async_copy(ctab_hbm.at[cidxv], crows, sem2)
            cp1.wait()
            cp2.wait()

            def edge(j, cc):
                for k in range(DH // 16):
                    sl = pl.ds(k * 16, 16)
                    rows[j, sl] = jnp.maximum(rows[j, sl] + crows[j, sl], 0.0)
                return cc
            lax.fori_loop(0, CH, edge, 0)
            pltpu.sync_copy(rows, aggr_sh.at[dstv], add=True)
            return carry
        lax.fori_loop(0, TRIPS, chunk, 0)
        plsc.subcore_barrier()

        @pl.when(s < NSUB - 1)
        def _():
            pltpu.sync_copy(aggr_sh.at[pl.ds(s * STR, STR)],
                            out_hbm.at[pl.ds(c * N + pbase + s * STR, STR)])

        @pl.when(s == NSUB - 1)
        def _():
            last = NH - (NSUB - 1) * STR
            pltpu.sync_copy(
                aggr_sh.at[pl.ds((NSUB - 1) * STR, last)],
                out_hbm.at[pl.ds(c * N + pbase + (NSUB - 1) * STR, last)])
        plsc.subcore_barrier()


@jax.jit
def _sc_aggregate(hi_flat, ctab_s, src, dst, cidx):
    mesh = plsc.VectorSubcoreMesh(core_axis_name="c", subcore_axis_name="s")
    f = pl.kernel(
        _sc_edge_body,
        out_type=jax.ShapeDtypeStruct((2 * N, DH), _f32),
        mesh=mesh,
        scratch_types=[
            pltpu.VMEM((CH,), jnp.int32),
            pltpu.VMEM((CH,), jnp.int32),
            pltpu.VMEM((CH,), jnp.int32),
            pltpu.VMEM((CH, DH), _f32),
            pltpu.VMEM((CH, DH), _f32),
            pltpu.VMEM((ZR, DH), _f32),
            pltpu.VMEM_SHARED((NPH, DH), _f32),
            pltpu.SemaphoreType.DMA,
            pltpu.SemaphoreType.DMA,
        ],
        compiler_params=pltpu.CompilerParams(use_tc_tiling_on_sc=False),
    )
    return f(hi_flat, ctab_s, src, dst, cidx)


def _node_spec():
    return pl.BlockSpec((2, RB, DH), lambda i: (0, i, 0))


def _full(shape):
    nd = len(shape)
    return pl.BlockSpec(shape, lambda i, _n=nd: (0,) * _n)


def _atom_body(x_ref, taba_ref, tabb_ref, out_ref):
    xb = x_ref[0]
    iot = lax.broadcasted_iota(jnp.int32, (RB, ATR), 1).astype(_f32)
    m = jnp.zeros((RB, ATR), _f32)
    for i in range(9):
        m = m + (iot == xb[:, i:i + 1] + float(AV * i)).astype(_f32)
    out_ref[0] = _dot(m, taba_ref[...])
    out_ref[1] = _dot(m, tabb_ref[...])


def _atom_encode(x3f, taba, tabb):
    return pl.pallas_call(
        _atom_body,
        grid=(NB,),
        in_specs=[pl.BlockSpec((1, RB, 9), lambda i: (i, 0, 0)),
                  _full((ATR, DH)), _full((ATR, DH))],
        out_specs=_node_spec(),
        out_shape=jax.ShapeDtypeStruct((2, N, DH), _f32),
    )(x3f, taba, tabb)


def _inject_body(h_ref, vn_ref, b_ref, out_ref):
    b = b_ref[0]
    ind = (lax.broadcasted_iota(jnp.int32, (RB, G), 1).astype(_f32)
           == b).astype(_f32)
    out_ref[0] = h_ref[0] + _dot(ind, vn_ref[0])
    out_ref[1] = h_ref[1] + _dot(ind, vn_ref[1])


def _inject(h, vn, bcolf):
    return pl.pallas_call(
        _inject_body,
        grid=(NB,),
        in_specs=[_node_spec(), _full((2, G, DH)),
                  pl.BlockSpec((1, RB, 1), lambda i: (i, 0, 0))],
        out_specs=_node_spec(),
        out_shape=jax.ShapeDtypeStruct((2, N, DH), _f32),
    )(h, vn, bcolf)


def _conv_body(do_relu, hi_ref, ag_ref, eps_ref, w1_ref, b1_ref, g1_ref,
               bb1_ref, w2_ref, b2_ref, g2_ref, bb2_ref, out_ref):
    e = 1.0 + eps_ref[0, 0]
    za = e * hi_ref[0] + ag_ref[0]
    zb = e * hi_ref[1] + ag_ref[1]
    z1 = _dot(za, w1_ref[0]) + _dot(zb, w1_ref[1]) + b1_ref[...]
    z1 = jnp.maximum(z1 * g1_ref[...] + bb1_ref[...], 0.0)
    ya = _dot(z1, w2_ref[0]) + b2_ref[0]
    ya = ya * g2_ref[0] + bb2_ref[0]
    yb = _dot(z1, w2_ref[1]) + b2_ref[1]
    yb = yb * g2_ref[1] + bb2_ref[1]
    if do_relu:
        ya = jnp.maximum(ya, 0.0)
        yb = jnp.maximum(yb, 0.0)
    out_ref[0] = ya
    out_ref[1] = yb


def _conv(hi, ag, epsl, w1, b1, g1, bb1, w2, b2, g2, bb2, do_relu):
    return pl.pallas_call(
        functools.partial(_conv_body, do_relu),
        grid=(NB,),
        in_specs=[_node_spec(), _node_spec(),
                  pl.BlockSpec(memory_space=pltpu.SMEM),
                  _full((2, DH, HP)), _full((1, HP)), _full((1, HP)),
                  _full((1, HP)), _full((2, HP, DH)), _full((2, 1, DH)),
                  _full((2, 1, DH)), _full((2, 1, DH))],
        out_specs=_node_spec(),
        out_shape=jax.ShapeDtypeStruct((2, N, DH), _f32),
    )(hi, ag, epsl, w1, b1, g1, bb1, w2, b2, g2, bb2)


def _vn_body(hi_ref, br_ref, vn_ref, vw1_ref, vb1_ref, vg1_ref, vbb1_ref,
             vw2_ref, vb2_ref, vg2_ref, vbb2_ref, out_ref, acc_ref):
    i = pl.program_id(0)

    @pl.when(i == 0)
    def _():
        acc_ref[...] = vn_ref[...]

    b = br_ref[0]
    ind = (lax.broadcasted_iota(jnp.int32, (G, RB), 0).astype(_f32)
           == b).astype(_f32)
    acc_ref[0] += _dot(ind, hi_ref[0])
    acc_ref[1] += _dot(ind, hi_ref[1])

    @pl.when(i == NB - 1)
    def _():
        vt = _dot(acc_ref[0], vw1_ref[0]) + _dot(acc_ref[1], vw1_ref[1])
        vt = jnp.maximum((vt + vb1_ref[...]) * vg1_ref[...] + vbb1_ref[...], 0.0)
        ya = (_dot(vt, vw2_ref[0]) + vb2_ref[0]) * vg2_ref[0] + vbb2_ref[0]
        yb = (_dot(vt, vw2_ref[1]) + vb2_ref[1]) * vg2_ref[1] + vbb2_ref[1]
        out_ref[0] = jnp.maximum(ya, 0.0)
        out_ref[1] = jnp.maximum(yb, 0.0)


def _vn_update(hi, browf, vn, vw1, vb1, vg1, vbb1, vw2, vb2, vg2, vbb2):
    return pl.pallas_call(
        _vn_body,
        grid=(NB,),
        in_specs=[_node_spec(),
                  pl.BlockSpec((1, 1, RB), lambda i: (i, 0, 0)),
                  _full((2, G, DH)), _full((2, DH, HP)), _full((1, HP)),
                  _full((1, HP)), _full((1, HP)), _full((2, HP, DH)),
                  _full((2, 1, DH)), _full((2, 1, DH)), _full((2, 1, DH))],
        out_specs=pl.BlockSpec((2, G, DH), lambda i: (0, 0, 0)),
        out_shape=jax.ShapeDtypeStruct((2, G, DH), _f32),
        scratch_shapes=[pltpu.VMEM((2, G, DH), _f32)],
    )(hi, browf, vn, vw1, vb1, vg1, vbb1, vw2, vb2, vg2, vbb2)


def _pool_body(h_ref, br_ref, pw_ref, pb_ref, out_ref, acc_ref, cnt_ref):
    i = pl.program_id(0)

    @pl.when(i == 0)
    def _():
        acc_ref[...] = jnp.zeros((2, G, DH), _f32)
        cnt_ref[...] = jnp.zeros((G, T), _f32)

    b = br_ref[0]
    ind = (lax.broadcasted_iota(jnp.int32, (G, RB), 0).astype(_f32)
           == b).astype(_f32)
    acc_ref[0] += _dot(ind, h_ref[0])
    acc_ref[1] += _dot(ind, h_ref[1])
    cnt_ref[...] += jnp.broadcast_to(jnp.sum(ind, axis=1, keepdims=True), (G, T))

    @pl.when(i == NB - 1)
    def _():
        inv = 1.0 / jnp.maximum(cnt_ref[:, 0:1], 1.0)
        ha = acc_ref[0] * inv
        hb = acc_ref[1] * inv
        out_ref[...] = _dot(ha, pw_ref[0]) + _dot(hb, pw_ref[1]) + pb_ref[...]


def _pool(h, browf, pw, pb):
    return pl.pallas_call(
        _pool_body,
        grid=(NB,),
        in_specs=[_node_spec(),
                  pl.BlockSpec((1, 1, RB), lambda i: (i, 0, 0)),
                  _full((2, DH, T)), _full((1, T))],
        out_specs=pl.BlockSpec((G, T), lambda i: (0, 0)),
        out_shape=jax.ShapeDtypeStruct((G, T), _f32),
        scratch_shapes=[pltpu.VMEM((2, G, DH), _f32),
                        pltpu.VMEM((G, T), _f32)],
    )(h, browf, pw, pb)


def _pad_to(a, shape):
    pads = [(0, t - s) for s, t in zip(a.shape, shape)]
    return jnp.pad(a.astype(_f32), pads)


def _halves_cols(a):
    p = _pad_to(a, a.shape[:-1] + (2 * DH,))
    return jnp.moveaxis(p.reshape(a.shape[:-1] + (2, DH)), -2, 0)


def _halves_rows(a, hp_cols):
    p = _pad_to(a, (2 * DH, hp_cols))
    return p.reshape(2, DH, hp_cols)


BNS = float((1.0 + 1e-5) ** -0.5)


def kernel(x, edge_index, edge_attr, batch, atom_emb, bond_emb, conv_eps,
           conv_W1, conv_b1, mlp_bn_g, mlp_bn_b, conv_W2, conv_b2, bn_g, bn_b,
           vn_emb0, vn_W1, vn_b1, vn_bn1_g, vn_bn1_b, vn_W2, vn_b2,
           vn_bn2_g, vn_bn2_b, pred_W, pred_b):
    x = x.astype(jnp.int32)
    ei = edge_index.astype(jnp.int32)
    ea = edge_attr.astype(jnp.int32)
    bat = batch.astype(jnp.int32)
    src, dst = ei[0], ei[1]
    cidx = ea[:, 0] * (BV * BV) + ea[:, 1] * BV + ea[:, 2]

    half = (dst >= NH).astype(jnp.int32)
    nothalf = 1 - half
    r0 = jnp.cumsum(nothalf) - nothalf
    r1 = jnp.cumsum(half) - half
    pos = jnp.where(half == 1, CAP + r1, r0)
    src = jnp.zeros((EPAD,), jnp.int32).at[pos].set(src)
    cidx = jnp.zeros((EPAD,), jnp.int32).at[pos].set(cidx)
    dst = jnp.full((EPAD,), 2 * N, jnp.int32).at[pos].set(dst)

    x3f = x.astype(_f32).reshape(NB, RB, 9)
    bcolf = bat.astype(_f32).reshape(NB, RB, 1)
    browf = bat.astype(_f32).reshape(NB, 1, RB)

    atab = _halves_cols(_pad_to(atom_emb.reshape(9 * AV, D), (ATR, D)))
    taba, tabb = atab[0], atab[1]

    ctab = (bond_emb[:, 0, :, None, None, :] + bond_emb[:, 1, None, :, None, :]
            + bond_emb[:, 2, None, None, :, :]).reshape(L, NC, D)
    ctab = _halves_cols(ctab)
    ctab_s = jnp.swapaxes(ctab, 0, 1).reshape(L, 2 * NC, DH)

    vn = _halves_cols(jnp.broadcast_to(vn_emb0[0], (G, D)))

    h = _atom_encode(x3f, taba, tabb)

    for l in range(L):
        w1 = _halves_rows(conv_W1[l], HP)
        b1 = _pad_to(conv_b1[l].reshape(1, H), (1, HP))
        g1 = _pad_to((mlp_bn_g[l] * BNS).reshape(1, H), (1, HP))
        bb1 = _pad_to(mlp_bn_b[l].reshape(1, H), (1, HP))
        w2 = _halves_cols(_pad_to(conv_W2[l], (HP, D)))
        b2 = _halves_cols(conv_b2[l].reshape(1, D))
        g2 = _halves_cols((bn_g[l] * BNS).reshape(1, D))
        bb2 = _halves_cols(bn_b[l].reshape(1, D))
        epsl = conv_eps[l].astype(_f32).reshape(1, 1)

        hi = _inject(h, vn, bcolf)
        aggr = _sc_aggregate(hi.reshape(2 * N, DH), ctab_s[l], src, dst,
                             cidx).reshape(2, N, DH)
        h = _conv(hi, aggr, epsl, w1, b1, g1, bb1, w2, b2, g2, bb2,
                  do_relu=(l < L - 1))

        if l < L - 1:
            vw1 = _halves_rows(vn_W1[l], HP)
            vb1 = _pad_to(vn_b1[l].reshape(1, H), (1, HP))
            vg1 = _pad_to((vn_bn1_g[l] * BNS).reshape(1, H), (1, HP))
            vbb1 = _pad_to(vn_bn1_b[l].reshape(1, H), (1, HP))
            vw2 = _halves_cols(_pad_to(vn_W2[l], (HP, D)))
            vb2 = _halves_cols(vn_b2[l].reshape(1, D))
            vg2 = _halves_cols((vn_bn2_g[l] * BNS).reshape(1, D))
            vbb2 = _halves_cols(vn_bn2_b[l].reshape(1, D))
            vn = _vn_update(hi, browf, vn, vw1, vb1, vg1, vbb1, vw2, vb2,
                            vg2, vbb2)

    pw = _halves_rows(pred_W, T)
    pb = pred_b.astype(_f32).reshape(1, T)
    return _pool(h, browf, pw, pb)


# speedup vs baseline: 1.7955x; 1.0115x over previous
import functools

import jax
import jax.numpy as jnp
from jax import lax
from jax.experimental import pallas as pl
from jax.experimental.pallas import tpu as pltpu
from jax.experimental.pallas import tpu_sc as plsc

N = 10000
E = 160000
G = 64
D = 300
H = 600
L = 5
T = 128
AV = 100
BV = 10

DH = 160
HP = 640
NB = 10
RB = N // NB
ATR = 1024
NC = 1000

CH = 80
NSUB = 16
NH = N // 2
TRIPS = 68
NCHP = NSUB * TRIPS
CAP = NCHP * CH
EPAD = 2 * CAP
STR = 320
NPH = NSUB * STR
GARB = NPH - 8
ZR = 160

_f32 = jnp.float32


def _dot(a, b):
    return jnp.dot(a, b, preferred_element_type=_f32)


def _sc_edge_body(hi_hbm, ctab_hbm, src_hbm, dst_hbm, cidx_hbm, out_hbm,
                  srcv, dstv, cidxv, rows, crows, zbuf, aggr_sh, ctab_sh,
                  sem1, sem2):
    c = lax.axis_index("c")
    s = lax.axis_index("s")

    def zfill(i, carry):
        for k in range(DH // 16):
            zbuf[i, pl.ds(k * 16, 16)] = jnp.zeros((16,), _f32)
        return carry
    lax.fori_loop(0, ZR, zfill, 0)

    coff_hi = c * N

    @pl.when(s < NSUB - 1)
    def _():
        pltpu.sync_copy(ctab_hbm.at[pl.ds(c * NC + s * 64, 64)],
                        ctab_sh.at[pl.ds(s * 64, 64)])

    @pl.when(s == NSUB - 1)
    def _():
        pltpu.sync_copy(ctab_hbm.at[pl.ds(c * NC + 960, NC - 960)],
                        ctab_sh.at[pl.ds(960, NC - 960)])
    plsc.subcore_barrier()

    for p in range(2):
        pltpu.sync_copy(zbuf, aggr_sh.at[pl.ds(s * STR, ZR)])
        pltpu.sync_copy(zbuf, aggr_sh.at[pl.ds(s * STR + ZR, STR - ZR)])
        plsc.subcore_barrier()

        pbase = p * NH

        def chunk(i, carry):
            base = (p * NCHP + s + i * NSUB) * CH
            pltpu.sync_copy(src_hbm.at[pl.ds(base, CH)], srcv)
            pltpu.sync_copy(cidx_hbm.at[pl.ds(base, CH)], cidxv)
            pltpu.sync_copy(dst_hbm.at[pl.ds(base, CH)], dstv)
            for k in range(CH // 16):
                sl = pl.ds(k * 16, 16)
                srcv[sl] = srcv[sl] + coff_hi
                d = dstv[sl] - pbase
                ok = (d >= 0) & (d < NH)
                dstv[sl] = jnp.where(ok, d, GARB)
            cp1 = pltpu.async_copy(hi_hbm.at[srcv], rows, sem1)
            pltpu.sync_copy(ctab_sh.at[cidxv], crows)
            cp1.wait()

            def edge(j, cc):
                for k in range(DH // 16):
                    sl = pl.ds(k * 16, 16)
                    rows[j, sl] = jnp.maximum(rows[j, sl] + crows[j, sl], 0.0)
                return cc
            lax.fori_loop(0, CH, edge, 0)
            pltpu.sync_copy(rows, aggr_sh.at[dstv], add=True)
            return carry
        lax.fori_loop(0, TRIPS, chunk, 0)
        plsc.subcore_barrier()

        @pl.when(s < NSUB - 1)
        def _():
            pltpu.sync_copy(aggr_sh.at[pl.ds(s * STR, STR)],
                            out_hbm.at[pl.ds(c * N + pbase + s * STR, STR)])

        @pl.when(s == NSUB - 1)
        def _():
            last = NH - (NSUB - 1) * STR
            pltpu.sync_copy(
                aggr_sh.at[pl.ds((NSUB - 1) * STR, last)],
                out_hbm.at[pl.ds(c * N + pbase + (NSUB - 1) * STR, last)])
        plsc.subcore_barrier()


@jax.jit
def _sc_aggregate(hi_flat, ctab_s, src, dst, cidx):
    mesh = plsc.VectorSubcoreMesh(core_axis_name="c", subcore_axis_name="s")
    f = pl.kernel(
        _sc_edge_body,
        out_type=jax.ShapeDtypeStruct((2 * N, DH), _f32),
        mesh=mesh,
        scratch_types=[
            pltpu.VMEM((CH,), jnp.int32),
            pltpu.VMEM((CH,), jnp.int32),
            pltpu.VMEM((CH,), jnp.int32),
            pltpu.VMEM((CH, DH), _f32),
            pltpu.VMEM((CH, DH), _f32),
            pltpu.VMEM((ZR, DH), _f32),
            pltpu.VMEM_SHARED((NPH, DH), _f32),
            pltpu.VMEM_SHARED((NC, DH), _f32),
            pltpu.SemaphoreType.DMA,
            pltpu.SemaphoreType.DMA,
        ],
        compiler_params=pltpu.CompilerParams(use_tc_tiling_on_sc=False),
    )
    return f(hi_flat, ctab_s, src, dst, cidx)


def _node_spec():
    return pl.BlockSpec((2, RB, DH), lambda i: (0, i, 0))


def _full(shape):
    nd = len(shape)
    return pl.BlockSpec(shape, lambda i, _n=nd: (0,) * _n)


def _atom_body(x_ref, taba_ref, tabb_ref, out_ref):
    xb = x_ref[0]
    iot = lax.broadcasted_iota(jnp.int32, (RB, ATR), 1).astype(_f32)
    m = jnp.zeros((RB, ATR), _f32)
    for i in range(9):
        m = m + (iot == xb[:, i:i + 1] + float(AV * i)).astype(_f32)
    out_ref[0] = _dot(m, taba_ref[...])
    out_ref[1] = _dot(m, tabb_ref[...])


def _atom_encode(x3f, taba, tabb):
    return pl.pallas_call(
        _atom_body,
        grid=(NB,),
        in_specs=[pl.BlockSpec((1, RB, 9), lambda i: (i, 0, 0)),
                  _full((ATR, DH)), _full((ATR, DH))],
        out_specs=_node_spec(),
        out_shape=jax.ShapeDtypeStruct((2, N, DH), _f32),
    )(x3f, taba, tabb)


def _inject_body(h_ref, vn_ref, b_ref, out_ref):
    b = b_ref[0]
    ind = (lax.broadcasted_iota(jnp.int32, (RB, G), 1).astype(_f32)
           == b).astype(_f32)
    out_ref[0] = h_ref[0] + _dot(ind, vn_ref[0])
    out_ref[1] = h_ref[1] + _dot(ind, vn_ref[1])


def _inject(h, vn, bcolf):
    return pl.pallas_call(
        _inject_body,
        grid=(NB,),
        in_specs=[_node_spec(), _full((2, G, DH)),
                  pl.BlockSpec((1, RB, 1), lambda i: (i, 0, 0))],
        out_specs=_node_spec(),
        out_shape=jax.ShapeDtypeStruct((2, N, DH), _f32),
    )(h, vn, bcolf)


def _conv_body(do_relu, hi_ref, ag_ref, eps_ref, w1_ref, b1_ref, g1_ref,
               bb1_ref, w2_ref, b2_ref, g2_ref, bb2_ref, out_ref):
    e = 1.0 + eps_ref[0, 0]
    za = e * hi_ref[0] + ag_ref[0]
    zb = e * hi_ref[1] + ag_ref[1]
    z1 = _dot(za, w1_ref[0]) + _dot(zb, w1_ref[1]) + b1_ref[...]
    z1 = jnp.maximum(z1 * g1_ref[...] + bb1_ref[...], 0.0)
    ya = _dot(z1, w2_ref[0]) + b2_ref[0]
    ya = ya * g2_ref[0] + bb2_ref[0]
    yb = _dot(z1, w2_ref[1]) + b2_ref[1]
    yb = yb * g2_ref[1] + bb2_ref[1]
    if do_relu:
        ya = jnp.maximum(ya, 0.0)
        yb = jnp.maximum(yb, 0.0)
    out_ref[0] = ya
    out_ref[1] = yb


def _conv(hi, ag, epsl, w1, b1, g1, bb1, w2, b2, g2, bb2, do_relu):
    return pl.pallas_call(
        functools.partial(_conv_body, do_relu),
        grid=(NB,),
        in_specs=[_node_spec(), _node_spec(),
                  pl.BlockSpec(memory_space=pltpu.SMEM),
                  _full((2, DH, HP)), _full((1, HP)), _full((1, HP)),
                  _full((1, HP)), _full((2, HP, DH)), _full((2, 1, DH)),
                  _full((2, 1, DH)), _full((2, 1, DH))],
        out_specs=_node_spec(),
        out_shape=jax.ShapeDtypeStruct((2, N, DH), _f32),
    )(hi, ag, epsl, w1, b1, g1, bb1, w2, b2, g2, bb2)


def _vn_body(hi_ref, br_ref, vn_ref, vw1_ref, vb1_ref, vg1_ref, vbb1_ref,
             vw2_ref, vb2_ref, vg2_ref, vbb2_ref, out_ref, acc_ref):
    i = pl.program_id(0)

    @pl.when(i == 0)
    def _():
        acc_ref[...] = vn_ref[...]

    b = br_ref[0]
    ind = (lax.broadcasted_iota(jnp.int32, (G, RB), 0).astype(_f32)
           == b).astype(_f32)
    acc_ref[0] += _dot(ind, hi_ref[0])
    acc_ref[1] += _dot(ind, hi_ref[1])

    @pl.when(i == NB - 1)
    def _():
        vt = _dot(acc_ref[0], vw1_ref[0]) + _dot(acc_ref[1], vw1_ref[1])
        vt = jnp.maximum((vt + vb1_ref[...]) * vg1_ref[...] + vbb1_ref[...], 0.0)
        ya = (_dot(vt, vw2_ref[0]) + vb2_ref[0]) * vg2_ref[0] + vbb2_ref[0]
        yb = (_dot(vt, vw2_ref[1]) + vb2_ref[1]) * vg2_ref[1] + vbb2_ref[1]
        out_ref[0] = jnp.maximum(ya, 0.0)
        out_ref[1] = jnp.maximum(yb, 0.0)


def _vn_update(hi, browf, vn, vw1, vb1, vg1, vbb1, vw2, vb2, vg2, vbb2):
    return pl.pallas_call(
        _vn_body,
        grid=(NB,),
        in_specs=[_node_spec(),
                  pl.BlockSpec((1, 1, RB), lambda i: (i, 0, 0)),
                  _full((2, G, DH)), _full((2, DH, HP)), _full((1, HP)),
                  _full((1, HP)), _full((1, HP)), _full((2, HP, DH)),
                  _full((2, 1, DH)), _full((2, 1, DH)), _full((2, 1, DH))],
        out_specs=pl.BlockSpec((2, G, DH), lambda i: (0, 0, 0)),
        out_shape=jax.ShapeDtypeStruct((2, G, DH), _f32),
        scratch_shapes=[pltpu.VMEM((2, G, DH), _f32)],
    )(hi, browf, vn, vw1, vb1, vg1, vbb1, vw2, vb2, vg2, vbb2)


def _pool_body(h_ref, br_ref, pw_ref, pb_ref, out_ref, acc_ref, cnt_ref):
    i = pl.program_id(0)

    @pl.when(i == 0)
    def _():
        acc_ref[...] = jnp.zeros((2, G, DH), _f32)
        cnt_ref[...] = jnp.zeros((G, T), _f32)

    b = br_ref[0]
    ind = (lax.broadcasted_iota(jnp.int32, (G, RB), 0).astype(_f32)
           == b).astype(_f32)
    acc_ref[0] += _dot(ind, h_ref[0])
    acc_ref[1] += _dot(ind, h_ref[1])
    cnt_ref[...] += jnp.broadcast_to(jnp.sum(ind, axis=1, keepdims=True), (G, T))

    @pl.when(i == NB - 1)
    def _():
        inv = 1.0 / jnp.maximum(cnt_ref[:, 0:1], 1.0)
        ha = acc_ref[0] * inv
        hb = acc_ref[1] * inv
        out_ref[...] = _dot(ha, pw_ref[0]) + _dot(hb, pw_ref[1]) + pb_ref[...]


def _pool(h, browf, pw, pb):
    return pl.pallas_call(
        _pool_body,
        grid=(NB,),
        in_specs=[_node_spec(),
                  pl.BlockSpec((1, 1, RB), lambda i: (i, 0, 0)),
                  _full((2, DH, T)), _full((1, T))],
        out_specs=pl.BlockSpec((G, T), lambda i: (0, 0)),
        out_shape=jax.ShapeDtypeStruct((G, T), _f32),
        scratch_shapes=[pltpu.VMEM((2, G, DH), _f32),
                        pltpu.VMEM((G, T), _f32)],
    )(h, browf, pw, pb)


def _pad_to(a, shape):
    pads = [(0, t - s) for s, t in zip(a.shape, shape)]
    return jnp.pad(a.astype(_f32), pads)


def _halves_cols(a):
    p = _pad_to(a, a.shape[:-1] + (2 * DH,))
    return jnp.moveaxis(p.reshape(a.shape[:-1] + (2, DH)), -2, 0)


def _halves_rows(a, hp_cols):
    p = _pad_to(a, (2 * DH, hp_cols))
    return p.reshape(2, DH, hp_cols)


BNS = float((1.0 + 1e-5) ** -0.5)


def kernel(x, edge_index, edge_attr, batch, atom_emb, bond_emb, conv_eps,
           conv_W1, conv_b1, mlp_bn_g, mlp_bn_b, conv_W2, conv_b2, bn_g, bn_b,
           vn_emb0, vn_W1, vn_b1, vn_bn1_g, vn_bn1_b, vn_W2, vn_b2,
           vn_bn2_g, vn_bn2_b, pred_W, pred_b):
    x = x.astype(jnp.int32)
    ei = edge_index.astype(jnp.int32)
    ea = edge_attr.astype(jnp.int32)
    bat = batch.astype(jnp.int32)
    src, dst = ei[0], ei[1]
    cidx = ea[:, 0] * (BV * BV) + ea[:, 1] * BV + ea[:, 2]

    half = (dst >= NH).astype(jnp.int32)
    nothalf = 1 - half
    r0 = jnp.cumsum(nothalf) - nothalf
    r1 = jnp.cumsum(half) - half
    pos = jnp.where(half == 1, CAP + r1, r0)
    src = jnp.zeros((EPAD,), jnp.int32).at[pos].set(src)
    cidx = jnp.zeros((EPAD,), jnp.int32).at[pos].set(cidx)
    dst = jnp.full((EPAD,), 2 * N, jnp.int32).at[pos].set(dst)

    x3f = x.astype(_f32).reshape(NB, RB, 9)
    bcolf = bat.astype(_f32).reshape(NB, RB, 1)
    browf = bat.astype(_f32).reshape(NB, 1, RB)

    atab = _halves_cols(_pad_to(atom_emb.reshape(9 * AV, D), (ATR, D)))
    taba, tabb = atab[0], atab[1]

    ctab = (bond_emb[:, 0, :, None, None, :] + bond_emb[:, 1, None, :, None, :]
            + bond_emb[:, 2, None, None, :, :]).reshape(L, NC, D)
    ctab = _halves_cols(ctab)
    ctab_s = jnp.swapaxes(ctab, 0, 1).reshape(L, 2 * NC, DH)

    vn = _halves_cols(jnp.broadcast_to(vn_emb0[0], (G, D)))

    h = _atom_encode(x3f, taba, tabb)

    for l in range(L):
        w1 = _halves_rows(conv_W1[l], HP)
        b1 = _pad_to(conv_b1[l].reshape(1, H), (1, HP))
        g1 = _pad_to((mlp_bn_g[l] * BNS).reshape(1, H), (1, HP))
        bb1 = _pad_to(mlp_bn_b[l].reshape(1, H), (1, HP))
        w2 = _halves_cols(_pad_to(conv_W2[l], (HP, D)))
        b2 = _halves_cols(conv_b2[l].reshape(1, D))
        g2 = _halves_cols((bn_g[l] * BNS).reshape(1, D))
        bb2 = _halves_cols(bn_b[l].reshape(1, D))
        epsl = conv_eps[l].astype(_f32).reshape(1, 1)

        hi = _inject(h, vn, bcolf)
        aggr = _sc_aggregate(hi.reshape(2 * N, DH), ctab_s[l], src, dst,
                             cidx).reshape(2, N, DH)
        h = _conv(hi, aggr, epsl, w1, b1, g1, bb1, w2, b2, g2, bb2,
                  do_relu=(l < L - 1))

        if l < L - 1:
            vw1 = _halves_rows(vn_W1[l], HP)
            vb1 = _pad_to(vn_b1[l].reshape(1, H), (1, HP))
            vg1 = _pad_to((vn_bn1_g[l] * BNS).reshape(1, H), (1, HP))
            vbb1 = _pad_to(vn_bn1_b[l].reshape(1, H), (1, HP))
            vw2 = _halves_cols(_pad_to(vn_W2[l], (HP, D)))
            vb2 = _halves_cols(vn_b2[l].reshape(1, D))
            vg2 = _halves_cols((vn_bn2_g[l] * BNS).reshape(1, D))
            vbb2 = _halves_cols(vn_bn2_b[l].reshape(1, D))
            vn = _vn_update(hi, browf, vn, vw1, vb1, vg1, vbb1, vw2, vb2,
                            vg2, vbb2)

    pw = _halves_rows(pred_W, T)
    pb = pred_b.astype(_f32).reshape(1, T)
    return _pool(h, browf, pw, pb)


# speedup vs baseline: 1.9061x; 1.0616x over previous
import functools

import jax
import jax.numpy as jnp
from jax import lax
from jax.experimental import pallas as pl
from jax.experimental.pallas import tpu as pltpu
from jax.experimental.pallas import tpu_sc as plsc

N = 10000
E = 160000
G = 64
D = 300
H = 600
L = 5
T = 128
AV = 100
BV = 10

DH = 160
HP = 640
NB = 10
RB = N // NB
ATR = 1024
NC = 1000

CH = 80
NSUB = 16
NH = N // 2
TRIPS = 68
NCHP = NSUB * TRIPS
CAP = NCHP * CH
EPAD = 2 * CAP
STR = 320
NPH = NSUB * STR
GARB = NPH - 8
ZR = 160

_f32 = jnp.float32


def _dot(a, b):
    return jnp.dot(a, b, preferred_element_type=_f32)


def _sc_edge_body(hi_hbm, ctab_hbm, src_hbm, dst_hbm, cidx_hbm, out_hbm,
                  srcv, dstv, cidxv, rows, crows,
                  dstv2, cidxv2, rows2, crows2, zbuf, aggr_sh,
                  sem1, sem2, sem3, sem4):
    c = lax.axis_index("c")
    s = lax.axis_index("s")

    def zfill(i, carry):
        for k in range(DH // 16):
            zbuf[i, pl.ds(k * 16, 16)] = jnp.zeros((16,), _f32)
        return carry
    lax.fori_loop(0, ZR, zfill, 0)

    coff_hi = c * N
    coff_ct = c * NC

    for p in range(2):
        pltpu.sync_copy(zbuf, aggr_sh.at[pl.ds(s * STR, ZR)])
        pltpu.sync_copy(zbuf, aggr_sh.at[pl.ds(s * STR + ZR, STR - ZR)])
        plsc.subcore_barrier()

        pbase = p * NH

        def issue(i, dv, cv, rv, crv, sem, semc):
            base = (p * NCHP + s + i * NSUB) * CH
            pltpu.sync_copy(src_hbm.at[pl.ds(base, CH)], srcv)
            pltpu.sync_copy(cidx_hbm.at[pl.ds(base, CH)], cv)
            pltpu.sync_copy(dst_hbm.at[pl.ds(base, CH)], dv)
            for k in range(CH // 16):
                sl = pl.ds(k * 16, 16)
                srcv[sl] = srcv[sl] + coff_hi
                cv[sl] = cv[sl] + coff_ct
                d = dv[sl] - pbase
                ok = (d >= 0) & (d < NH)
                dv[sl] = jnp.where(ok, d, GARB)
            return (pltpu.async_copy(hi_hbm.at[srcv], rv, sem),
                    pltpu.async_copy(ctab_hbm.at[cv], crv, semc))

        def finish(dv, rv, crv):
            def edge(j, cc):
                for k in range(DH // 16):
                    sl = pl.ds(k * 16, 16)
                    rv[j, sl] = jnp.maximum(rv[j, sl] + crv[j, sl], 0.0)
                return cc
            lax.fori_loop(0, CH, edge, 0)
            pltpu.sync_copy(rv, aggr_sh.at[dv], add=True)

        def chunk(t, carry):
            cpa1, cpa2 = issue(2 * t, dstv, cidxv, rows, crows, sem1, sem2)
            cpb1, cpb2 = issue(2 * t + 1, dstv2, cidxv2, rows2, crows2,
                               sem3, sem4)
            cpa1.wait()
            cpa2.wait()
            finish(dstv, rows, crows)
            cpb1.wait()
            cpb2.wait()
            finish(dstv2, rows2, crows2)
            return carry
        lax.fori_loop(0, TRIPS // 2, chunk, 0)
        plsc.subcore_barrier()

        @pl.when(s < NSUB - 1)
        def _():
            pltpu.sync_copy(aggr_sh.at[pl.ds(s * STR, STR)],
                            out_hbm.at[pl.ds(c * N + pbase + s * STR, STR)])

        @pl.when(s == NSUB - 1)
        def _():
            last = NH - (NSUB - 1) * STR
            pltpu.sync_copy(
                aggr_sh.at[pl.ds((NSUB - 1) * STR, last)],
                out_hbm.at[pl.ds(c * N + pbase + (NSUB - 1) * STR, last)])
        plsc.subcore_barrier()


@jax.jit
def _sc_aggregate(hi_flat, ctab_s, src, dst, cidx):
    mesh = plsc.VectorSubcoreMesh(core_axis_name="c", subcore_axis_name="s")
    f = pl.kernel(
        _sc_edge_body,
        out_type=jax.ShapeDtypeStruct((2 * N, DH), _f32),
        mesh=mesh,
        scratch_types=[
            pltpu.VMEM((CH,), jnp.int32),
            pltpu.VMEM((CH,), jnp.int32),
            pltpu.VMEM((CH,), jnp.int32),
            pltpu.VMEM((CH, DH), _f32),
            pltpu.VMEM((CH, DH), _f32),
            pltpu.VMEM((CH,), jnp.int32),
            pltpu.VMEM((CH,), jnp.int32),
            pltpu.VMEM((CH, DH), _f32),
            pltpu.VMEM((CH, DH), _f32),
            pltpu.VMEM((ZR, DH), _f32),
            pltpu.VMEM_SHARED((NPH, DH), _f32),
            pltpu.SemaphoreType.DMA,
            pltpu.SemaphoreType.DMA,
            pltpu.SemaphoreType.DMA,
            pltpu.SemaphoreType.DMA,
        ],
        compiler_params=pltpu.CompilerParams(use_tc_tiling_on_sc=False),
    )
    return f(hi_flat, ctab_s, src, dst, cidx)


def _node_spec():
    return pl.BlockSpec((2, RB, DH), lambda i: (0, i, 0))


def _full(shape):
    nd = len(shape)
    return pl.BlockSpec(shape, lambda i, _n=nd: (0,) * _n)


def _atom_body(x_ref, taba_ref, tabb_ref, out_ref):
    xb = x_ref[0]
    iot = lax.broadcasted_iota(jnp.int32, (RB, ATR), 1).astype(_f32)
    m = jnp.zeros((RB, ATR), _f32)
    for i in range(9):
        m = m + (iot == xb[:, i:i + 1] + float(AV * i)).astype(_f32)
    out_ref[0] = _dot(m, taba_ref[...])
    out_ref[1] = _dot(m, tabb_ref[...])


def _atom_encode(x3f, taba, tabb):
    return pl.pallas_call(
        _atom_body,
        grid=(NB,),
        in_specs=[pl.BlockSpec((1, RB, 9), lambda i: (i, 0, 0)),
                  _full((ATR, DH)), _full((ATR, DH))],
        out_specs=_node_spec(),
        out_shape=jax.ShapeDtypeStruct((2, N, DH), _f32),
    )(x3f, taba, tabb)


def _inject_body(h_ref, vn_ref, b_ref, out_ref):
    b = b_ref[0]
    ind = (lax.broadcasted_iota(jnp.int32, (RB, G), 1).astype(_f32)
           == b).astype(_f32)
    out_ref[0] = h_ref[0] + _dot(ind, vn_ref[0])
    out_ref[1] = h_ref[1] + _dot(ind, vn_ref[1])


def _inject(h, vn, bcolf):
    return pl.pallas_call(
        _inject_body,
        grid=(NB,),
        in_specs=[_node_spec(), _full((2, G, DH)),
                  pl.BlockSpec((1, RB, 1), lambda i: (i, 0, 0))],
        out_specs=_node_spec(),
        out_shape=jax.ShapeDtypeStruct((2, N, DH), _f32),
    )(h, vn, bcolf)


def _conv_body(do_relu, hi_ref, ag_ref, eps_ref, w1_ref, b1_ref, g1_ref,
               bb1_ref, w2_ref, b2_ref, g2_ref, bb2_ref, out_ref):
    e = 1.0 + eps_ref[0, 0]
    za = e * hi_ref[0] + ag_ref[0]
    zb = e * hi_ref[1] + ag_ref[1]
    z1 = _dot(za, w1_ref[0]) + _dot(zb, w1_ref[1]) + b1_ref[...]
    z1 = jnp.maximum(z1 * g1_ref[...] + bb1_ref[...], 0.0)
    ya = _dot(z1, w2_ref[0]) + b2_ref[0]
    ya = ya * g2_ref[0] + bb2_ref[0]
    yb = _dot(z1, w2_ref[1]) + b2_ref[1]
    yb = yb * g2_ref[1] + bb2_ref[1]
    if do_relu:
        ya = jnp.maximum(ya, 0.0)
        yb = jnp.maximum(yb, 0.0)
    out_ref[0] = ya
    out_ref[1] = yb


def _conv(hi, ag, epsl, w1, b1, g1, bb1, w2, b2, g2, bb2, do_relu):
    return pl.pallas_call(
        functools.partial(_conv_body, do_relu),
        grid=(NB,),
        in_specs=[_node_spec(), _node_spec(),
                  pl.BlockSpec(memory_space=pltpu.SMEM),
                  _full((2, DH, HP)), _full((1, HP)), _full((1, HP)),
                  _full((1, HP)), _full((2, HP, DH)), _full((2, 1, DH)),
                  _full((2, 1, DH)), _full((2, 1, DH))],
        out_specs=_node_spec(),
        out_shape=jax.ShapeDtypeStruct((2, N, DH), _f32),
    )(hi, ag, epsl, w1, b1, g1, bb1, w2, b2, g2, bb2)


def _vn_body(hi_ref, br_ref, vn_ref, vw1_ref, vb1_ref, vg1_ref, vbb1_ref,
             vw2_ref, vb2_ref, vg2_ref, vbb2_ref, out_ref, acc_ref):
    i = pl.program_id(0)

    @pl.when(i == 0)
    def _():
        acc_ref[...] = vn_ref[...]

    b = br_ref[0]
    ind = (lax.broadcasted_iota(jnp.int32, (G, RB), 0).astype(_f32)
           == b).astype(_f32)
    acc_ref[0] += _dot(ind, hi_ref[0])
    acc_ref[1] += _dot(ind, hi_ref[1])

    @pl.when(i == NB - 1)
    def _():
        vt = _dot(acc_ref[0], vw1_ref[0]) + _dot(acc_ref[1], vw1_ref[1])
        vt = jnp.maximum((vt + vb1_ref[...]) * vg1_ref[...] + vbb1_ref[...], 0.0)
        ya = (_dot(vt, vw2_ref[0]) + vb2_ref[0]) * vg2_ref[0] + vbb2_ref[0]
        yb = (_dot(vt, vw2_ref[1]) + vb2_ref[1]) * vg2_ref[1] + vbb2_ref[1]
        out_ref[0] = jnp.maximum(ya, 0.0)
        out_ref[1] = jnp.maximum(yb, 0.0)


def _vn_update(hi, browf, vn, vw1, vb1, vg1, vbb1, vw2, vb2, vg2, vbb2):
    return pl.pallas_call(
        _vn_body,
        grid=(NB,),
        in_specs=[_node_spec(),
                  pl.BlockSpec((1, 1, RB), lambda i: (i, 0, 0)),
                  _full((2, G, DH)), _full((2, DH, HP)), _full((1, HP)),
                  _full((1, HP)), _full((1, HP)), _full((2, HP, DH)),
                  _full((2, 1, DH)), _full((2, 1, DH)), _full((2, 1, DH))],
        out_specs=pl.BlockSpec((2, G, DH), lambda i: (0, 0, 0)),
        out_shape=jax.ShapeDtypeStruct((2, G, DH), _f32),
        scratch_shapes=[pltpu.VMEM((2, G, DH), _f32)],
    )(hi, browf, vn, vw1, vb1, vg1, vbb1, vw2, vb2, vg2, vbb2)


def _pool_body(h_ref, br_ref, pw_ref, pb_ref, out_ref, acc_ref, cnt_ref):
    i = pl.program_id(0)

    @pl.when(i == 0)
    def _():
        acc_ref[...] = jnp.zeros((2, G, DH), _f32)
        cnt_ref[...] = jnp.zeros((G, T), _f32)

    b = br_ref[0]
    ind = (lax.broadcasted_iota(jnp.int32, (G, RB), 0).astype(_f32)
           == b).astype(_f32)
    acc_ref[0] += _dot(ind, h_ref[0])
    acc_ref[1] += _dot(ind, h_ref[1])
    cnt_ref[...] += jnp.broadcast_to(jnp.sum(ind, axis=1, keepdims=True), (G, T))

    @pl.when(i == NB - 1)
    def _():
        inv = 1.0 / jnp.maximum(cnt_ref[:, 0:1], 1.0)
        ha = acc_ref[0] * inv
        hb = acc_ref[1] * inv
        out_ref[...] = _dot(ha, pw_ref[0]) + _dot(hb, pw_ref[1]) + pb_ref[...]


def _pool(h, browf, pw, pb):
    return pl.pallas_call(
        _pool_body,
        grid=(NB,),
        in_specs=[_node_spec(),
                  pl.BlockSpec((1, 1, RB), lambda i: (i, 0, 0)),
                  _full((2, DH, T)), _full((1, T))],
        out_specs=pl.BlockSpec((G, T), lambda i: (0, 0)),
        out_shape=jax.ShapeDtypeStruct((G, T), _f32),
        scratch_shapes=[pltpu.VMEM((2, G, DH), _f32),
                        pltpu.VMEM((G, T), _f32)],
    )(h, browf, pw, pb)


def _pad_to(a, shape):
    pads = [(0, t - s) for s, t in zip(a.shape, shape)]
    return jnp.pad(a.astype(_f32), pads)


def _halves_cols(a):
    p = _pad_to(a, a.shape[:-1] + (2 * DH,))
    return jnp.moveaxis(p.reshape(a.shape[:-1] + (2, DH)), -2, 0)


def _halves_rows(a, hp_cols):
    p = _pad_to(a, (2 * DH, hp_cols))
    return p.reshape(2, DH, hp_cols)


BNS = float((1.0 + 1e-5) ** -0.5)


def kernel(x, edge_index, edge_attr, batch, atom_emb, bond_emb, conv_eps,
           conv_W1, conv_b1, mlp_bn_g, mlp_bn_b, conv_W2, conv_b2, bn_g, bn_b,
           vn_emb0, vn_W1, vn_b1, vn_bn1_g, vn_bn1_b, vn_W2, vn_b2,
           vn_bn2_g, vn_bn2_b, pred_W, pred_b):
    x = x.astype(jnp.int32)
    ei = edge_index.astype(jnp.int32)
    ea = edge_attr.astype(jnp.int32)
    bat = batch.astype(jnp.int32)
    src, dst = ei[0], ei[1]
    cidx = ea[:, 0] * (BV * BV) + ea[:, 1] * BV + ea[:, 2]

    half = (dst >= NH).astype(jnp.int32)
    nothalf = 1 - half
    r0 = jnp.cumsum(nothalf) - nothalf
    r1 = jnp.cumsum(half) - half
    pos = jnp.where(half == 1, CAP + r1, r0)
    src = jnp.zeros((EPAD,), jnp.int32).at[pos].set(src)
    cidx = jnp.zeros((EPAD,), jnp.int32).at[pos].set(cidx)
    dst = jnp.full((EPAD,), 2 * N, jnp.int32).at[pos].set(dst)

    x3f = x.astype(_f32).reshape(NB, RB, 9)
    bcolf = bat.astype(_f32).reshape(NB, RB, 1)
    browf = bat.astype(_f32).reshape(NB, 1, RB)

    atab = _halves_cols(_pad_to(atom_emb.reshape(9 * AV, D), (ATR, D)))
    taba, tabb = atab[0], atab[1]

    ctab = (bond_emb[:, 0, :, None, None, :] + bond_emb[:, 1, None, :, None, :]
            + bond_emb[:, 2, None, None, :, :]).reshape(L, NC, D)
    ctab = _halves_cols(ctab)
    ctab_s = jnp.swapaxes(ctab, 0, 1).reshape(L, 2 * NC, DH)

    vn = _halves_cols(jnp.broadcast_to(vn_emb0[0], (G, D)))

    h = _atom_encode(x3f, taba, tabb)

    for l in range(L):
        w1 = _halves_rows(conv_W1[l], HP)
        b1 = _pad_to(conv_b1[l].reshape(1, H), (1, HP))
        g1 = _pad_to((mlp_bn_g[l] * BNS).reshape(1, H), (1, HP))
        bb1 = _pad_to(mlp_bn_b[l].reshape(1, H), (1, HP))
        w2 = _halves_cols(_pad_to(conv_W2[l], (HP, D)))
        b2 = _halves_cols(conv_b2[l].reshape(1, D))
        g2 = _halves_cols((bn_g[l] * BNS).reshape(1, D))
        bb2 = _halves_cols(bn_b[l].reshape(1, D))
        epsl = conv_eps[l].astype(_f32).reshape(1, 1)

        hi = _inject(h, vn, bcolf)
        aggr = _sc_aggregate(hi.reshape(2 * N, DH), ctab_s[l], src, dst,
                             cidx).reshape(2, N, DH)
        h = _conv(hi, aggr, epsl, w1, b1, g1, bb1, w2, b2, g2, bb2,
                  do_relu=(l < L - 1))

        if l < L - 1:
            vw1 = _halves_rows(vn_W1[l], HP)
            vb1 = _pad_to(vn_b1[l].reshape(1, H), (1, HP))
            vg1 = _pad_to((vn_bn1_g[l] * BNS).reshape(1, H), (1, HP))
            vbb1 = _pad_to(vn_bn1_b[l].reshape(1, H), (1, HP))
            vw2 = _halves_cols(_pad_to(vn_W2[l], (HP, D)))
            vb2 = _halves_cols(vn_b2[l].reshape(1, D))
            vg2 = _halves_cols((vn_bn2_g[l] * BNS).reshape(1, D))
            vbb2 = _halves_cols(vn_bn2_b[l].reshape(1, D))
            vn = _vn_update(hi, browf, vn, vw1, vb1, vg1, vbb1, vw2, vb2,
                            vg2, vbb2)

    pw = _halves_rows(pred_W, T)
    pb = pred_b.astype(_f32).reshape(1, T)
    return _pool(h, browf, pw, pb)
